# baseline (speedup 1.0000x reference)
import functools

import jax
import jax.numpy as jnp
from jax import lax
from jax.experimental import pallas as pl
from jax.experimental.pallas import tpu as pltpu
from jax.experimental.pallas import tpu_sc as plsc

VOCAB = 1000000
EMBED_DIM = 32
BATCH = 16384

_NUM_CORES = 2
_NUM_SUBCORES = 16
_NW = _NUM_CORES * _NUM_SUBCORES
_B_PER_W = BATCH // _NW

_mesh = plsc.VectorSubcoreMesh(core_axis_name="c", subcore_axis_name="s")


@functools.partial(
    pl.kernel,
    mesh=_mesh,
    out_type=jax.ShapeDtypeStruct((EMBED_DIM, BATCH), jnp.float32),
    scratch_types=[
        pltpu.VMEM((_B_PER_W,), jnp.int32),
        pltpu.VMEM((_B_PER_W,), jnp.int32),
        pltpu.VMEM((_B_PER_W,), jnp.int32),
        pltpu.VMEM((_B_PER_W, 128), jnp.float32),
        pltpu.VMEM((EMBED_DIM, _B_PER_W), jnp.float32),
        pltpu.SemaphoreType.DMA,
    ],
    compiler_params=pltpu.CompilerParams(needs_layout_passes=False),
)
def _gather_kernel(
    idx_hbm, table_hbm, out_hbm, idx_v, row_v, sub_v, rows_v, out_v, sem
):
    wid = lax.axis_index("s") * _NUM_CORES + lax.axis_index("c")
    base = wid * _B_PER_W
    pltpu.sync_copy(idx_hbm.at[pl.ds(base, _B_PER_W)], idx_v)

    lane = lax.iota(jnp.int32, 16)

    def split(t, carry):
        v = idx_v[pl.ds(16 * t, 16)]
        row_v[pl.ds(16 * t, 16)] = lax.shift_right_logical(v, 2)
        sub_v[pl.ds(16 * t, 16)] = (v & 3) * 32
        return carry

    lax.fori_loop(0, _B_PER_W // 16, split, 0)

    pltpu.async_copy(table_hbm.at[row_v], rows_v, sem).wait()

    def extract(t, carry):
        kvec = 16 * t + lane
        cvec = sub_v[pl.ds(16 * t, 16)]
        for j in range(EMBED_DIM):
            out_v[j, pl.ds(16 * t, 16)] = plsc.load_gather(
                rows_v, [kvec, cvec + j]
            )
        return carry

    lax.fori_loop(0, _B_PER_W // 16, extract, 0)

    pltpu.sync_copy(out_v, out_hbm.at[:, pl.ds(base, _B_PER_W)])


def kernel(in_tensor, table):
    idx = in_tensor.reshape(BATCH).astype(jnp.int32)
    table128 = table.reshape(VOCAB // 4, 128)
    out_t = _gather_kernel(idx, table128)
    return out_t.T


# speedup vs baseline: 1.0060x; 1.0060x over previous
import functools

import jax
import jax.numpy as jnp
from jax import lax
from jax.experimental import pallas as pl
from jax.experimental.pallas import tpu as pltpu
from jax.experimental.pallas import tpu_sc as plsc

VOCAB = 1000000
EMBED_DIM = 32
BATCH = 16384

_NUM_CORES = 2
_NUM_SUBCORES = 16
_NW = _NUM_CORES * _NUM_SUBCORES
_B_PER_W = BATCH // _NW

_mesh = plsc.VectorSubcoreMesh(core_axis_name="c", subcore_axis_name="s")


@functools.partial(
    pl.kernel,
    mesh=_mesh,
    out_type=jax.ShapeDtypeStruct((BATCH, EMBED_DIM), jnp.float32),
    scratch_types=[
        pltpu.VMEM((_B_PER_W,), jnp.int32),
        pltpu.VMEM((_B_PER_W, EMBED_DIM), jnp.float32),
        pltpu.SemaphoreType.DMA,
    ],
    compiler_params=pltpu.CompilerParams(use_tc_tiling_on_sc=False),
)
def _gather_kernel(idx_hbm, table_hbm, out_hbm, idx_v, rows_v, sem):
    wid = lax.axis_index("s") * _NUM_CORES + lax.axis_index("c")
    base = wid * _B_PER_W
    pltpu.sync_copy(idx_hbm.at[pl.ds(base, _B_PER_W)], idx_v)
    pltpu.async_copy(table_hbm.at[idx_v], rows_v, sem).wait()
    pltpu.sync_copy(rows_v, out_hbm.at[pl.ds(base, _B_PER_W)])


def kernel(in_tensor, table):
    idx = in_tensor.reshape(BATCH).astype(jnp.int32)
    return _gather_kernel(idx, table)
